# baseline (speedup 1.0000x reference)
import dataclasses
import functools

import jax
import jax.numpy as jnp
from jax import lax
from jax.experimental import pallas as pl
from jax.experimental.pallas import tpu as pltpu
from jax.experimental.pallas import tpu_sc as plsc

N = 10000
E = 320000
D = 128
H = 128
C = 40
GP_PARA = 10.0
DA_PARA = 1.0
NEG_SLOPE = 0.01

NP_ = 10240
PADROW = 10200
NC = 2
NS = 16
NW = NC * NS
EB = 128
EPT32 = 10112
E_PAD = NW * EPT32
EPT16 = E_PAD // NS
NBLK = EPT16 // EB
RPT = NP_ // NS

_vmesh = plsc.VectorSubcoreMesh(core_axis_name="c", subcore_axis_name="s")

_sc_cp = pltpu.CompilerParams()
if "needs_layout_passes" in pltpu.CompilerParams.__dataclass_fields__:
    _sc_cp = dataclasses.replace(_sc_cp, needs_layout_passes=False)


def _sc_degrees(src_s, dst_s, src_t, dst_t):
    @functools.partial(
        pl.kernel,
        mesh=_vmesh,
        compiler_params=_sc_cp,
        out_type=jax.ShapeDtypeStruct((4, NW, NP_), jnp.float32),
        scratch_types=[
            pltpu.VMEM((NP_,), jnp.float32),
            pltpu.VMEM((EPT32,), jnp.int32),
        ],
    )
    def k(ss_ref, ds_ref, st_ref, dt_ref, out_ref, hist, idxbuf):
        cid = lax.axis_index("c")
        sid = lax.axis_index("s")
        wid = cid * NS + sid
        ones = jnp.full((16,), 1.0, jnp.float32)
        for a, arr in enumerate((ss_ref, ds_ref, st_ref, dt_ref)):
            @pl.loop(0, NP_, step=16)
            def _(j):
                hist.at[pl.ds(j, 16)][...] = jnp.zeros((16,), jnp.float32)

            pltpu.sync_copy(arr.at[wid], idxbuf)

            @pl.loop(0, EPT32, step=16)
            def _(j):
                v = idxbuf.at[pl.ds(j, 16)][...]
                plsc.addupdate_scatter(hist, [v], ones)

            pltpu.sync_copy(hist, out_ref.at[a, wid])

    return k(src_s, dst_s, src_t, dst_t)


WINDOWS = ((0, 3456), (3456, 3456), (6912, 3328))
ACC_R = 3456
ZR = 72


def _sc_scatter(p_s, p_t, src3_s, dst3_s, src3_t, dst3_t):
    out1 = jax.ShapeDtypeStruct((NP_, D), jnp.float32)

    @functools.partial(
        pl.kernel,
        mesh=_vmesh,
        compiler_params=_sc_cp,
        out_type=[out1, out1],
        scratch_types=[
            pltpu.VMEM((NBLK, EB), jnp.int32),
            pltpu.VMEM((NBLK, EB), jnp.int32),
            pltpu.VMEM((EB,), jnp.int32),
            pltpu.VMEM((EB,), jnp.int32),
            pltpu.VMEM((EB,), jnp.int32),
            pltpu.VMEM((EB,), jnp.int32),
            pltpu.VMEM((EB, D), jnp.float32),
            pltpu.VMEM((EB, D), jnp.float32),
            pltpu.VMEM((ZR, D), jnp.float32),
            pltpu.VMEM_SHARED((ACC_R, D), jnp.float32),
            pltpu.SemaphoreType.DMA,
            pltpu.SemaphoreType.DMA,
        ],
    )
    def k(ps_ref, pt_ref, ss_ref, ds_ref, st_ref, dt_ref, qs_ref, qt_ref,
          sidx, didx, s2a, d2a, s2b, d2b, rows0, rows1, zblk, acc,
          sem0, sem1):
        cid = lax.axis_index("c")
        sid = lax.axis_index("s")

        @pl.loop(0, ZR)
        def _(r):
            @pl.loop(0, D, step=16)
            def _(c0):
                zblk.at[r, pl.ds(c0, 16)][...] = jnp.zeros((16,), jnp.float32)

        def one_pass(p_ref, out_ref, base, win):
            rpt = win // NS

            def remap(g, s2, d2):
                @pl.loop(0, EB, step=16)
                def _(j):
                    sv = sidx.at[g, pl.ds(j, 16)][...]
                    dv = didx.at[g, pl.ds(j, 16)][...] - base
                    ok = (dv >= 0) & (dv < win)
                    s2.at[pl.ds(j, 16)][...] = jnp.where(
                        ok, sv, jnp.full((16,), PADROW, jnp.int32))
                    d2.at[pl.ds(j, 16)][...] = jnp.where(ok, dv, sv % win)

            nz72, rem = divmod(rpt, ZR)
            for z in range(nz72):
                pltpu.sync_copy(zblk, acc.at[pl.ds(sid * rpt + z * ZR, ZR)])
            if rem:
                pltpu.sync_copy(zblk.at[pl.ds(0, rem)],
                                acc.at[pl.ds(sid * rpt + nz72 * ZR, rem)])

            plsc.subcore_barrier()

            remap(0, s2a, d2a)
            pltpu.async_copy(p_ref.at[s2a], rows0, sem0)

            @pl.loop(0, NBLK, step=2)
            def _(g):
                remap(g + 1, s2b, d2b)
                pltpu.make_async_copy(p_ref.at[s2a], rows0, sem0).wait()
                pltpu.async_copy(p_ref.at[s2b], rows1, sem1)
                pltpu.sync_copy(rows0, acc.at[d2a], add=True)

                @pl.when(g + 2 < NBLK)
                def _():
                    remap(g + 2, s2a, d2a)

                pltpu.make_async_copy(p_ref.at[s2b], rows1, sem1).wait()

                @pl.when(g + 2 < NBLK)
                def _():
                    pltpu.async_copy(p_ref.at[s2a], rows0, sem0)

                pltpu.sync_copy(rows1, acc.at[d2b], add=True)

            plsc.subcore_barrier()
            pltpu.sync_copy(acc.at[pl.ds(sid * rpt, rpt)],
                            out_ref.at[pl.ds(base + sid * rpt, rpt)])

        def run(p_ref, s3, d3, out_ref):
            pltpu.sync_copy(s3.at[sid], sidx)
            pltpu.sync_copy(d3.at[sid], didx)
            for base, win in WINDOWS:
                one_pass(p_ref, out_ref, base, win)
                plsc.subcore_barrier()

        @pl.when(cid == 0)
        def _():
            run(ps_ref, ss_ref, ds_ref, qs_ref)

        @pl.when(cid == 1)
        def _():
            run(pt_ref, st_ref, dt_ref, qt_ref)

    return k(p_s, p_t, src3_s, dst3_s, src3_t, dst3_t)


def _tc_dn(hists):
    def body(h_ref, o_ref):
        deg = jnp.sum(h_ref[...], axis=1)
        o_ref[...] = lax.rsqrt(jnp.maximum(deg, 1.0))

    return pl.pallas_call(
        body, out_shape=jax.ShapeDtypeStruct((4, NP_), jnp.float32))(hists)


def _rmask():
    return lax.broadcasted_iota(jnp.int32, (NP_, 1), 0) < N


def _tc_mm1(x_s, x_t, W1, b1r, dno_s, dno_t):
    def body(xs_ref, xt_ref, w_ref, b_ref, dns_ref, dnt_ref, ps_ref, pt_ref):
        w = w_ref[...].astype(jnp.bfloat16)
        b = b_ref[...]
        rmask = _rmask()
        for x_ref, dn_ref, p_ref in ((xs_ref, dns_ref, ps_ref),
                                     (xt_ref, dnt_ref, pt_ref)):
            h = lax.dot_general(x_ref[...].astype(jnp.bfloat16), w,
                                (((1,), (0,)), ((), ())),
                                preferred_element_type=jnp.float32)
            p_ref[...] = jnp.where(rmask, (h + b) * dn_ref[...], 0.0)

    return pl.pallas_call(
        body,
        out_shape=[jax.ShapeDtypeStruct((NP_, H), jnp.float32),
                   jax.ShapeDtypeStruct((NP_, H), jnp.float32)],
    )(x_s, x_t, W1, b1r, dno_s, dno_t)


def _tc_mm2(q_s, q_t, W2, b2r, dni_s, dni_t, dno_s, dno_t):
    def body(qs_ref, qt_ref, w_ref, b_ref, dis_ref, dit_ref, dos_ref, dot_ref,
             ps_ref, pt_ref):
        w = w_ref[...].astype(jnp.bfloat16)
        b = b_ref[...]
        rmask = _rmask()
        for q_ref, di_ref, do_ref, p_ref in (
                (qs_ref, dis_ref, dos_ref, ps_ref),
                (qt_ref, dit_ref, dot_ref, pt_ref)):
            v = q_ref[...] * di_ref[...]
            h1 = jnp.where(v >= 0.0, v, v * NEG_SLOPE)
            h = lax.dot_general(h1.astype(jnp.bfloat16), w,
                                (((1,), (0,)), ((), ())),
                                preferred_element_type=jnp.float32)
            p_ref[...] = jnp.where(rmask, (h + b) * do_ref[...], 0.0)

    return pl.pallas_call(
        body,
        out_shape=[jax.ShapeDtypeStruct((NP_, H), jnp.float32),
                   jax.ShapeDtypeStruct((NP_, H), jnp.float32)],
    )(q_s, q_t, W2, b2r, dni_s, dni_t, dno_s, dno_t)


def _tc_final(q2_s, q2_t, dni_s, dni_t, fcWp, fcbp, labels_p, wvec):
    def body(qs_ref, qt_ref, dis_ref, dit_ref, w_ref, b_ref, lab_ref, wv_ref,
             out_ref):
        rmask = _rmask()
        colv = lax.broadcasted_iota(jnp.int32, (1, 128), 1) < C
        col_iota = lax.broadcasted_iota(jnp.int32, (NP_, 128), 1)

        vs = qs_ref[...] * dis_ref[...]
        hs = jnp.where(rmask & (vs >= 0.0), vs, jnp.where(rmask, vs * NEG_SLOPE, 0.0))
        vt = qt_ref[...] * dit_ref[...]
        ht = jnp.where(rmask & (vt >= 0.0), vt, jnp.where(rmask, vt * NEG_SLOPE, 0.0))

        fcw = w_ref[...].astype(jnp.bfloat16)
        fs = lax.dot_general(hs.astype(jnp.bfloat16), fcw,
                             (((1,), (0,)), ((), ())),
                             preferred_element_type=jnp.float32) + b_ref[...]

        fsm = jnp.where(colv, fs, -1e30)
        rowmax = jnp.max(fsm, axis=1, keepdims=True)
        ex = jnp.where(colv, jnp.exp(fsm - rowmax), 0.0)
        lse = jnp.log(jnp.sum(ex, axis=1, keepdims=True)) + rowmax
        flab = jnp.sum(jnp.where(lab_ref[...] == col_iota, fs, 0.0),
                       axis=1, keepdims=True)
        nll = jnp.sum(jnp.where(rmask, lse - flab, 0.0)) * (1.0 / N)

        colmean_fs = jnp.sum(jnp.where(rmask & colv, fs, 0.0), axis=0,
                             keepdims=True) * (1.0 / N)
        colmean_ht = jnp.sum(ht, axis=0, keepdims=True) * (1.0 / N)
        colmean_ft = lax.dot_general(colmean_ht, w_ref[...],
                                     (((1,), (0,)), ((), ())),
                                     preferred_element_type=jnp.float32) \
            + b_ref[...]

        wv = wv_ref[...]
        wd = jnp.sum((colmean_fs - colmean_ft) * wv)
        gl = (jnp.sqrt(jnp.sum(wv * wv)) - 1.0) ** 2
        gp = -wd + GP_PARA * gl
        da = nll + DA_PARA * wd

        row8 = lax.broadcasted_iota(jnp.int32, (8, 128), 0)
        out_ref[...] = jnp.where(row8 == 0, gp, jnp.where(row8 == 1, da, 0.0))

    return pl.pallas_call(
        body, out_shape=jax.ShapeDtypeStruct((8, 128), jnp.float32),
    )(q2_s, q2_t, dni_s, dni_t, fcWp, fcbp, labels_p, wvec)


def kernel(features_s, labels_s, features_t, edge_index_s, edge_index_t,
           W1, b1, W2, b2, fc_W, fc_b, disc_W, disc_b):
    def pad_idx(a):
        return jnp.concatenate(
            [a.astype(jnp.int32), jnp.full((E_PAD - E,), PADROW, jnp.int32)])

    src_s = pad_idx(edge_index_s[0])
    dst_s = pad_idx(edge_index_s[1])
    src_t = pad_idx(edge_index_t[0])
    dst_t = pad_idx(edge_index_t[1])

    hists = _sc_degrees(src_s.reshape(NW, EPT32), dst_s.reshape(NW, EPT32),
                        src_t.reshape(NW, EPT32), dst_t.reshape(NW, EPT32))
    dn = _tc_dn(hists)
    dno_s = dn[0].reshape(NP_, 1)
    dni_s = dn[1].reshape(NP_, 1)
    dno_t = dn[2].reshape(NP_, 1)
    dni_t = dn[3].reshape(NP_, 1)

    zpad = jnp.zeros((NP_ - N, D), jnp.float32)
    x_s = jnp.concatenate([features_s, zpad])
    x_t = jnp.concatenate([features_t, zpad])

    s3_s = src_s.reshape(NS, NBLK, EB)
    d3_s = dst_s.reshape(NS, NBLK, EB)
    s3_t = src_t.reshape(NS, NBLK, EB)
    d3_t = dst_t.reshape(NS, NBLK, EB)

    b1r = b1.reshape(1, H)
    b2r = b2.reshape(1, H)

    p1 = _tc_mm1(x_s, x_t, W1, b1r, dno_s, dno_t)
    q1 = _sc_scatter(*p1, s3_s, d3_s, s3_t, d3_t)
    p2 = _tc_mm2(*q1, W2, b2r, dni_s, dni_t, dno_s, dno_t)
    q2 = _sc_scatter(*p2, s3_s, d3_s, s3_t, d3_t)

    fcWp = jnp.zeros((H, 128), jnp.float32).at[:, :C].set(fc_W)
    fcbp = jnp.zeros((1, 128), jnp.float32).at[:, :C].set(fc_b.reshape(1, C))
    wvec = jnp.zeros((1, 128), jnp.float32).at[:, :C].set(disc_W.reshape(1, C))
    labels_p = jnp.concatenate(
        [labels_s.astype(jnp.int32), jnp.zeros((NP_ - N,), jnp.int32)]
    ).reshape(NP_, 1)

    out = _tc_final(*q2, dni_s, dni_t, fcWp, fcbp, labels_p, wvec)
    return (out[0, 0], out[1, 0])
